# baseline (speedup 1.0000x reference)
import jax
import jax.numpy as jnp
from jax.experimental import pallas as pl
from jax.experimental.pallas import tpu as pltpu

_N_TILE = 1000


def _agg_kernel(v_ref, nbr_ref, w_ref, b_ref, out_ref):
    kf = jnp.float32(nbr_ref.shape[0])
    s = jnp.sum(nbr_ref[...], axis=0)
    x = jnp.concatenate([v_ref[...] * kf, s], axis=1)
    out_ref[...] = (
        jnp.dot(x, w_ref[...], preferred_element_type=jnp.float32)
        + kf * b_ref[...]
    )


def kernel(v, neighbors, W, b):
    N, D = v.shape
    K = neighbors.shape[0]
    grid = (N // _N_TILE,)
    return pl.pallas_call(
        _agg_kernel,
        grid=grid,
        in_specs=[
            pl.BlockSpec((_N_TILE, D), lambda i: (i, 0)),
            pl.BlockSpec((K, _N_TILE, D), lambda i: (0, i, 0)),
            pl.BlockSpec((2 * D, D), lambda i: (0, 0)),
            pl.BlockSpec((1, D), lambda i: (0, 0)),
        ],
        out_specs=pl.BlockSpec((_N_TILE, D), lambda i: (i, 0)),
        out_shape=jax.ShapeDtypeStruct((N, D), jnp.float32),
        compiler_params=pltpu.CompilerParams(
            dimension_semantics=("arbitrary",),
        ),
    )(v, neighbors, W, b.reshape(1, D))


# speedup vs baseline: 1.0326x; 1.0326x over previous
import jax
import jax.numpy as jnp
from jax.experimental import pallas as pl
from jax.experimental.pallas import tpu as pltpu

_N_TILE = 400


def _agg_kernel(v_ref, nbr_ref, w_ref, b_ref, out_ref):
    kf = jnp.float32(nbr_ref.shape[0])
    s = jnp.sum(nbr_ref[...], axis=0)
    x = jnp.concatenate([v_ref[...] * kf, s], axis=1)
    out_ref[...] = (
        jnp.dot(x, w_ref[...], preferred_element_type=jnp.float32)
        + kf * b_ref[...]
    )


def kernel(v, neighbors, W, b):
    N, D = v.shape
    K = neighbors.shape[0]
    grid = (N // _N_TILE,)
    return pl.pallas_call(
        _agg_kernel,
        grid=grid,
        in_specs=[
            pl.BlockSpec((_N_TILE, D), lambda i: (i, 0)),
            pl.BlockSpec((K, _N_TILE, D), lambda i: (0, i, 0)),
            pl.BlockSpec((2 * D, D), lambda i: (0, 0)),
            pl.BlockSpec((1, D), lambda i: (0, 0)),
        ],
        out_specs=pl.BlockSpec((_N_TILE, D), lambda i: (i, 0)),
        out_shape=jax.ShapeDtypeStruct((N, D), jnp.float32),
        compiler_params=pltpu.CompilerParams(
            dimension_semantics=("arbitrary",),
        ),
    )(v, neighbors, W, b.reshape(1, D))


# speedup vs baseline: 1.0357x; 1.0030x over previous
import jax
import jax.numpy as jnp
from jax.experimental import pallas as pl
from jax.experimental.pallas import tpu as pltpu

_N_TILE = 400


_N_SPLIT = 4


def _agg_kernel(v_ref, *rest):
    nbr_refs = rest[:_N_SPLIT]
    w_ref, b_ref, out_ref = rest[_N_SPLIT:]
    kf = jnp.float32(sum(r.shape[0] for r in nbr_refs))
    parts = [jnp.sum(r[...], axis=0) for r in nbr_refs]
    while len(parts) > 1:
        parts = [a + b for a, b in zip(parts[::2], parts[1::2])]
    x = jnp.concatenate([v_ref[...] * kf, parts[0]], axis=1)
    out_ref[...] = (
        jnp.dot(x, w_ref[...], preferred_element_type=jnp.float32)
        + kf * b_ref[...]
    )


def kernel(v, neighbors, W, b):
    N, D = v.shape
    K = neighbors.shape[0]
    kc = K // _N_SPLIT
    grid = (N // _N_TILE,)
    nbr_specs = [
        pl.BlockSpec((kc, _N_TILE, D), lambda i, j=j: (j, i, 0))
        for j in range(_N_SPLIT)
    ]
    return pl.pallas_call(
        _agg_kernel,
        grid=grid,
        in_specs=[pl.BlockSpec((_N_TILE, D), lambda i: (i, 0))]
        + nbr_specs
        + [
            pl.BlockSpec((2 * D, D), lambda i: (0, 0)),
            pl.BlockSpec((1, D), lambda i: (0, 0)),
        ],
        out_specs=pl.BlockSpec((_N_TILE, D), lambda i: (i, 0)),
        out_shape=jax.ShapeDtypeStruct((N, D), jnp.float32),
        compiler_params=pltpu.CompilerParams(
            dimension_semantics=("arbitrary",),
        ),
    )(v, *([neighbors] * _N_SPLIT), W, b.reshape(1, D))
